# Initial kernel scaffold; baseline (speedup 1.0000x reference)
import jax
import jax.numpy as jnp
from jax.experimental import pallas as pl


def kernel(en0, en1, en2, de0, de1, de2, global_step):
    raise NotImplementedError("write your pallas kernel here")


# speedup vs baseline: 1.5696x; 1.5696x over previous
import functools

import jax
import jax.numpy as jnp
from jax.experimental import pallas as pl
from jax.experimental.pallas import tpu as pltpu

Q = 5.0
WARMUP = 200
MINK = 100

H = 128
W = 128
B = 16
N_TOTAL = B * H * W
K_TOP = max(MINK, int(N_TOTAL * Q / 100.0))

_INT_MIN = -2147483648
_INT_MAXP = 0x7FFFFFFF


def _dist_body(en_ref, de_ref, out_ref, aa, ab, bb):
    j = pl.program_id(1)
    nj = pl.num_programs(1)
    a = en_ref[0]
    b = de_ref[0]
    paa = jnp.sum(a * a, axis=0)
    pab = jnp.sum(a * b, axis=0)
    pbb = jnp.sum(b * b, axis=0)

    @pl.when(j == 0)
    def _():
        aa[...] = paa
        ab[...] = pab
        bb[...] = pbb

    @pl.when(j > 0)
    def _():
        aa[...] += paa
        ab[...] += pab
        bb[...] += pbb

    @pl.when(j == nj - 1)
    def _():
        na = jnp.maximum(jnp.sqrt(aa[...]), 1e-8)
        nb = jnp.maximum(jnp.sqrt(bb[...]), 1e-8)
        out_ref[0] = 1.0 - ab[...] / (na * nb)


def _dist_map(en, de, cb):
    c = en.shape[1]
    grid = (B, c // cb)
    spec = pl.BlockSpec((1, cb, H, W), lambda i, j: (i, j, 0, 0))
    return pl.pallas_call(
        _dist_body,
        grid=grid,
        in_specs=[spec, spec],
        out_specs=pl.BlockSpec((1, H, W), lambda i, j: (i, 0, 0)),
        out_shape=jax.ShapeDtypeStruct((B, H, W), jnp.float32),
        scratch_shapes=[pltpu.VMEM((H, W), jnp.float32)] * 3,
    )(en, de)


def _select_body(d0_ref, d1_ref, d2_ref, out_ref, v_ref, s_ref):
    v = (d0_ref[...] + d1_ref[...] + d2_ref[...]) * (1.0 / 3.0)
    v_ref[...] = v
    bits = jax.lax.bitcast_convert_type(v, jnp.int32)
    s = bits ^ (jax.lax.shift_right_arithmetic(bits, 31) & jnp.int32(_INT_MAXP))
    s_ref[...] = s

    kk = jnp.int32(K_TOP)

    def body(i, prefix_u):
        bit = jax.lax.shift_left(jnp.int32(1), jnp.int32(31) - i)
        cand_u = prefix_u | bit
        thresh_s = cand_u ^ jnp.int32(_INT_MIN)
        cnt = jnp.sum((s_ref[...] >= thresh_s).astype(jnp.int32))
        return jnp.where(cnt >= kk, cand_u, prefix_u)

    prefix_u = jax.lax.fori_loop(0, 32, body, jnp.int32(0))
    t_s = prefix_u ^ jnp.int32(_INT_MIN)
    t_bits = jnp.where(t_s >= 0, t_s, t_s ^ jnp.int32(_INT_MAXP))
    t_val = jax.lax.bitcast_convert_type(t_bits, jnp.float32)

    sdat = s_ref[...]
    gt = sdat > t_s
    cnt_gt = jnp.sum(gt.astype(jnp.float32))
    sum_gt = jnp.sum(jnp.where(gt, v_ref[...], 0.0))
    out_ref[0, 0] = (sum_gt + (jnp.float32(K_TOP) - cnt_gt) * t_val) * (
        1.0 / K_TOP
    )


def _select(d0, d1, d2):
    rows = N_TOTAL // W
    spec = pl.BlockSpec((rows, W), lambda: (0, 0))
    return pl.pallas_call(
        _select_body,
        in_specs=[spec, spec, spec],
        out_specs=pl.BlockSpec(memory_space=pltpu.SMEM),
        out_shape=jax.ShapeDtypeStruct((1, 1), jnp.float32),
        scratch_shapes=[
            pltpu.VMEM((rows, W), jnp.float32),
            pltpu.VMEM((rows, W), jnp.int32),
        ],
    )(
        d0.reshape(rows, W),
        d1.reshape(rows, W),
        d2.reshape(rows, W),
    )


def kernel(en0, en1, en2, de0, de1, de2, global_step):
    d0 = _dist_map(en0, de0, 96)
    d1 = _dist_map(en1, de1, 96)
    d2 = _dist_map(en2, de2, 96)
    topk_mean = _select(d0, d1, d2)[0, 0]
    progress = global_step / WARMUP
    warm = 100.0 - (100.0 - Q) * progress
    q_current = jnp.where(global_step < WARMUP, warm, Q).astype(jnp.float32)
    return topk_mean * (q_current / Q)
